# Initial kernel scaffold; baseline (speedup 1.0000x reference)
import jax
import jax.numpy as jnp
from jax import lax
from jax.experimental import pallas as pl
from jax.experimental.pallas import tpu as pltpu
from jax.experimental.pallas import tpu_sc as plsc

VOCAB = 100000
MAX_LEN = 200
HIDDEN = 128
BATCH = 1024
N_TOK = BATCH * MAX_LEN
NW = 32
TOK_PER_W = N_TOK // NW
CHUNK = 128
NCHUNK = TOK_PER_W // CHUNK
PAIRS = NCHUNK // 2
NJ = HIDDEN // 16


def _sc_body(tok_table, combo, tok_idx, cmb_idx, gamma, beta, out,
             tok_idx_v, cmb_idx_v, buf, cbuf, obuf, gv, bv,
             gsem, csem, osem):
  wid = lax.axis_index("s") * 2 + lax.axis_index("c")
  w_base = wid * TOK_PER_W

  pltpu.sync_copy(gamma, gv)
  pltpu.sync_copy(beta, bv)
  gvs = [gv[pl.ds(16 * j, 16)] for j in range(NJ)]
  bvs = [bv[pl.ds(16 * j, 16)] for j in range(NJ)]

  def issue_gathers(g, s):
    base = w_base + g * CHUNK
    pltpu.sync_copy(tok_idx.at[pl.ds(base, CHUNK)], tok_idx_v.at[s])
    pltpu.sync_copy(cmb_idx.at[pl.ds(base, CHUNK)], cmb_idx_v.at[s])
    pltpu.async_copy(tok_table.at[tok_idx_v.at[s]], buf.at[s], gsem.at[s])
    pltpu.async_copy(combo.at[cmb_idx_v.at[s]], cbuf.at[s], csem.at[s])

  def wait_gathers(s):
    pltpu.make_async_copy(tok_table.at[tok_idx_v.at[s]], buf.at[s],
                          gsem.at[s]).wait()
    pltpu.make_async_copy(combo.at[cmb_idx_v.at[s]], cbuf.at[s],
                          csem.at[s]).wait()

  def out_copy(g, s):
    base = w_base + g * CHUNK
    return pltpu.make_async_copy(obuf.at[s], out.at[pl.ds(base, CHUNK)],
                                 osem.at[s])

  def compute(s):
    @plsc.parallel_loop(0, CHUNK, 1, unroll=4)
    def _(t):
      x = [buf[s, t, pl.ds(16 * j, 16)] + cbuf[s, t, pl.ds(16 * j, 16)]
           for j in range(NJ)]
      tot = ((x[0] + x[1]) + (x[2] + x[3])) + ((x[4] + x[5]) + (x[6] + x[7]))
      mean = jnp.sum(tot) * (1.0 / HIDDEN)
      d = [xj - mean for xj in x]
      sq = [dj * dj for dj in d]
      v = ((sq[0] + sq[1]) + (sq[2] + sq[3])) + ((sq[4] + sq[5]) + (sq[6] + sq[7]))
      a = jnp.broadcast_to(jnp.sum(v) * (1.0 / HIDDEN) + 1e-5, (16,))
      yi = jnp.int32(0x5F3759DF) - (plsc.bitcast(a, jnp.int32) >> 1)
      y = plsc.bitcast(yi, jnp.float32)
      h = a * 0.5
      for _ in range(3):
        y = y * (1.5 - h * y * y)
      for j in range(NJ):
        obuf[s, t, pl.ds(16 * j, 16)] = d[j] * y * gvs[j] + bvs[j]

  issue_gathers(0, 0)
  issue_gathers(1, 1)

  def pair_body(p, carry):
    for s in (0, 1):
      g = 2 * p + s

      @pl.when(p >= 1)
      def _():
        out_copy(g - 2, s).wait()

      wait_gathers(s)
      compute(s)
      out_copy(g, s).start()

      @pl.when(p < PAIRS - 1)
      def _():
        issue_gathers(g + 2, s)
    return carry

  lax.fori_loop(0, PAIRS, pair_body, 0)
  out_copy(NCHUNK - 2, 0).wait()
  out_copy(NCHUNK - 1, 1).wait()


_sc_call = pl.kernel(
    _sc_body,
    out_type=jax.ShapeDtypeStruct((N_TOK, HIDDEN), jnp.float32),
    mesh=plsc.VectorSubcoreMesh(core_axis_name="c", subcore_axis_name="s"),
    scratch_types=[
        pltpu.VMEM((2, CHUNK), jnp.int32),
        pltpu.VMEM((2, CHUNK), jnp.int32),
        pltpu.VMEM((2, CHUNK, HIDDEN), jnp.float32),
        pltpu.VMEM((2, CHUNK, HIDDEN), jnp.float32),
        pltpu.VMEM((2, CHUNK, HIDDEN), jnp.float32),
        pltpu.VMEM((HIDDEN,), jnp.float32),
        pltpu.VMEM((HIDDEN,), jnp.float32),
        pltpu.SemaphoreType.DMA((2,)),
        pltpu.SemaphoreType.DMA((2,)),
        pltpu.SemaphoreType.DMA((2,)),
    ],
)


def kernel(input_token, segment_ids, token_table, type_table, pos_table,
           gamma, beta):
  tok_idx = input_token.reshape(-1)
  cmb_idx = (2 * jnp.arange(MAX_LEN, dtype=jnp.int32)[None, :]
             + segment_ids).reshape(-1)
  combo = (pos_table[:, None, :] + type_table[None, :, :]).reshape(
      2 * MAX_LEN, HIDDEN)
  out = _sc_call(token_table, combo, tok_idx, cmb_idx, gamma, beta)
  return out.reshape(BATCH, MAX_LEN, HIDDEN)


# speedup vs baseline: 6.5918x; 6.5918x over previous
import jax
import jax.numpy as jnp
from jax import lax
from jax.experimental import pallas as pl
from jax.experimental.pallas import tpu as pltpu
from jax.experimental.pallas import tpu_sc as plsc

VOCAB = 100000
MAX_LEN = 200
HIDDEN = 128
BATCH = 1024
N_TOK = BATCH * MAX_LEN
NW = 32
TOK_PER_W = N_TOK // NW
CHUNK = 128
NCHUNK = TOK_PER_W // CHUNK
PAIRS = NCHUNK // 2
NJ = HIDDEN // 16


def _sc_body(tok_table, combo, tok_idx, cmb_idx, gamma, beta, out,
             tok_idx_v, cmb_idx_v, buf, cbuf, obuf, gv, bv,
             ybuf, sbuf, s2buf, mbuf, rbuf,
             gsem, csem, osem):
  wid = lax.axis_index("s") * 2 + lax.axis_index("c")
  w_base = wid * TOK_PER_W

  pltpu.sync_copy(gamma, gv)
  pltpu.sync_copy(beta, bv)
  gvs = [gv[pl.ds(16 * j, 16)] for j in range(NJ)]
  bvs = [bv[pl.ds(16 * j, 16)] for j in range(NJ)]

  lanes = lax.iota(jnp.int32, 16)

  def issue_gathers(g, s):
    base = w_base + g * CHUNK
    pltpu.sync_copy(tok_idx.at[pl.ds(base, CHUNK)], tok_idx_v.at[s])
    pltpu.sync_copy(cmb_idx.at[pl.ds(base, CHUNK)], cmb_idx_v.at[s])
    pltpu.async_copy(tok_table.at[tok_idx_v.at[s]], buf.at[s], gsem.at[s])
    pltpu.async_copy(combo.at[cmb_idx_v.at[s]], cbuf.at[s], csem.at[s])

  def wait_gathers(s):
    pltpu.make_async_copy(tok_table.at[tok_idx_v.at[s]], buf.at[s],
                          gsem.at[s]).wait()
    pltpu.make_async_copy(combo.at[cmb_idx_v.at[s]], cbuf.at[s],
                          csem.at[s]).wait()

  def out_copy(g, s):
    base = w_base + g * CHUNK
    return pltpu.make_async_copy(obuf.at[s], out.at[pl.ds(base, CHUNK)],
                                 osem.at[s])

  def compute(s):
    def group(grp, carry):
      @plsc.parallel_loop(0, 16, 1, unroll=2)
      def _(ti):
        t = grp * 16 + ti
        y = [buf[s, t, pl.ds(16 * j, 16)] + cbuf[s, t, pl.ds(16 * j, 16)]
             for j in range(NJ)]
        tot = ((y[0] + y[1]) + (y[2] + y[3])) + ((y[4] + y[5]) + (y[6] + y[7]))
        q = [yj * yj for yj in y]
        sq = ((q[0] + q[1]) + (q[2] + q[3])) + ((q[4] + q[5]) + (q[6] + q[7]))
        for j in range(NJ):
          ybuf[ti, pl.ds(16 * j, 16)] = y[j]
        sbuf[pl.ds(ti * 17, 16)] = tot
        s2buf[pl.ds(ti * 17, 16)] = sq

      zero = jnp.zeros((16,), jnp.float32)

      @plsc.parallel_loop(0, 16, 1, unroll=4,
                          carry=(lanes * 17, zero, zero))
      def red(k, c):
        ck, acc, acc2 = c
        acc = acc + plsc.load_gather(sbuf, (ck,))
        acc2 = acc2 + plsc.load_gather(s2buf, (ck,))
        return ck + 1, acc, acc2

      _, acc, acc2 = red
      mean = acc * (1.0 / HIDDEN)
      var = acc2 * (1.0 / HIDDEN) - mean * mean
      a = var + 1e-5
      yi = jnp.int32(0x5F3759DF) - (plsc.bitcast(a, jnp.int32) >> 1)
      r = plsc.bitcast(yi, jnp.float32)
      h = a * 0.5
      for _ in range(3):
        r = r * (1.5 - h * r * r)
      mbuf[:] = mean
      rbuf[:] = r

      @plsc.parallel_loop(0, 16, 1, unroll=2)
      def _(ti):
        t = grp * 16 + ti
        tsplat = jnp.full((16,), ti, jnp.int32)
        m = plsc.load_gather(mbuf, (tsplat,))
        rs = plsc.load_gather(rbuf, (tsplat,))
        for j in range(NJ):
          yj = ybuf[ti, pl.ds(16 * j, 16)]
          obuf[s, t, pl.ds(16 * j, 16)] = (yj - m) * rs * gvs[j] + bvs[j]
      return carry

    lax.fori_loop(0, CHUNK // 16, group, 0)

  issue_gathers(0, 0)
  issue_gathers(1, 1)

  def pair_body(p, carry):
    for s in (0, 1):
      g = 2 * p + s

      @pl.when(p >= 1)
      def _():
        out_copy(g - 2, s).wait()

      wait_gathers(s)
      compute(s)
      out_copy(g, s).start()

      @pl.when(p < PAIRS - 1)
      def _():
        issue_gathers(g + 2, s)
    return carry

  lax.fori_loop(0, PAIRS, pair_body, 0)
  out_copy(NCHUNK - 2, 0).wait()
  out_copy(NCHUNK - 1, 1).wait()


_sc_call = pl.kernel(
    _sc_body,
    out_type=jax.ShapeDtypeStruct((N_TOK, HIDDEN), jnp.float32),
    mesh=plsc.VectorSubcoreMesh(core_axis_name="c", subcore_axis_name="s"),
    compiler_params=pltpu.CompilerParams(needs_layout_passes=False),
    scratch_types=[
        pltpu.VMEM((2, CHUNK), jnp.int32),
        pltpu.VMEM((2, CHUNK), jnp.int32),
        pltpu.VMEM((2, CHUNK, HIDDEN), jnp.float32),
        pltpu.VMEM((2, CHUNK, HIDDEN), jnp.float32),
        pltpu.VMEM((2, CHUNK, HIDDEN), jnp.float32),
        pltpu.VMEM((HIDDEN,), jnp.float32),
        pltpu.VMEM((HIDDEN,), jnp.float32),
        pltpu.VMEM((16, HIDDEN), jnp.float32),
        pltpu.VMEM((16 * 17,), jnp.float32),
        pltpu.VMEM((16 * 17,), jnp.float32),
        pltpu.VMEM((16,), jnp.float32),
        pltpu.VMEM((16,), jnp.float32),
        pltpu.SemaphoreType.DMA((2,)),
        pltpu.SemaphoreType.DMA((2,)),
        pltpu.SemaphoreType.DMA((2,)),
    ],
)


def kernel(input_token, segment_ids, token_table, type_table, pos_table,
           gamma, beta):
  tok_idx = input_token.reshape(-1)
  cmb_idx = (2 * jnp.arange(MAX_LEN, dtype=jnp.int32)[None, :]
             + segment_ids).reshape(-1)
  combo = (pos_table[:, None, :] + type_table[None, :, :]).reshape(
      2 * MAX_LEN, HIDDEN)
  out = _sc_call(token_table, combo, tok_idx, cmb_idx, gamma, beta)
  return out.reshape(BATCH, MAX_LEN, HIDDEN)


# speedup vs baseline: 6.7816x; 1.0288x over previous
import jax
import jax.numpy as jnp
from jax import lax
from jax.experimental import pallas as pl
from jax.experimental.pallas import tpu as pltpu
from jax.experimental.pallas import tpu_sc as plsc

VOCAB = 100000
MAX_LEN = 200
HIDDEN = 128
BATCH = 1024
N_TOK = BATCH * MAX_LEN
NW = 32
TOK_PER_W = N_TOK // NW
CHUNK = 128
NCHUNK = TOK_PER_W // CHUNK
TRIPLES = (NCHUNK - 2) // 3
NJ = HIDDEN // 16


def _sc_body(tok_table, combo, tok_idx, cmb_idx, gamma, beta, out,
             tok_idx_v, cmb_idx_v, buf, obuf, gv, bv,
             ybuf, sbuf, s2buf, mbuf, rbuf,
             tsem, asem, osem):
  wid = lax.axis_index("s") * 2 + lax.axis_index("c")
  w_base = wid * TOK_PER_W

  pltpu.sync_copy(gamma, gv)
  pltpu.sync_copy(beta, bv)
  gvs = [gv[pl.ds(16 * j, 16)] for j in range(NJ)]
  bvs = [bv[pl.ds(16 * j, 16)] for j in range(NJ)]

  lanes = lax.iota(jnp.int32, 16)

  def issue_tok(g, s):
    base = w_base + g * CHUNK
    pltpu.sync_copy(tok_idx.at[pl.ds(base, CHUNK)], tok_idx_v.at[s])
    pltpu.sync_copy(cmb_idx.at[pl.ds(base, CHUNK)], cmb_idx_v.at[s])
    pltpu.async_copy(tok_table.at[tok_idx_v.at[s]], buf.at[s], tsem.at[s])

  def wait_tok(s):
    pltpu.make_async_copy(tok_table.at[tok_idx_v.at[s]], buf.at[s],
                          tsem.at[s]).wait()

  def issue_add(s):
    pltpu.async_copy(combo.at[cmb_idx_v.at[s]], buf.at[s], asem.at[s],
                     add=True)

  def wait_add(s):
    pltpu.make_async_copy(combo.at[cmb_idx_v.at[s]], buf.at[s],
                          asem.at[s]).wait()

  def out_copy(g, s):
    base = w_base + g * CHUNK
    return pltpu.make_async_copy(obuf.at[s], out.at[pl.ds(base, CHUNK)],
                                 osem.at[s])

  def compute(s):
    def group(grp, carry):
      @plsc.parallel_loop(0, 16, 1, unroll=2)
      def _(ti):
        t = grp * 16 + ti
        y = [buf[s, t, pl.ds(16 * j, 16)] for j in range(NJ)]
        tot = ((y[0] + y[1]) + (y[2] + y[3])) + ((y[4] + y[5]) + (y[6] + y[7]))
        q = [yj * yj for yj in y]
        sq = ((q[0] + q[1]) + (q[2] + q[3])) + ((q[4] + q[5]) + (q[6] + q[7]))
        for j in range(NJ):
          ybuf[ti, pl.ds(16 * j, 16)] = y[j]
        sbuf[pl.ds(ti * 17, 16)] = tot
        s2buf[pl.ds(ti * 17, 16)] = sq

      zero = jnp.zeros((16,), jnp.float32)

      @plsc.parallel_loop(0, 16, 1, unroll=4,
                          carry=(lanes * 17, zero, zero))
      def red(k, c):
        ck, acc, acc2 = c
        acc = acc + plsc.load_gather(sbuf, (ck,))
        acc2 = acc2 + plsc.load_gather(s2buf, (ck,))
        return ck + 1, acc, acc2

      _, acc, acc2 = red
      mean = acc * (1.0 / HIDDEN)
      var = acc2 * (1.0 / HIDDEN) - mean * mean
      a = var + 1e-5
      yi = jnp.int32(0x5F3759DF) - (plsc.bitcast(a, jnp.int32) >> 1)
      r = plsc.bitcast(yi, jnp.float32)
      h = a * 0.5
      for _ in range(3):
        r = r * (1.5 - h * r * r)
      mbuf[:] = mean
      rbuf[:] = r

      @plsc.parallel_loop(0, 16, 1, unroll=2)
      def _(ti):
        t = grp * 16 + ti
        tsplat = jnp.full((16,), ti, jnp.int32)
        m = plsc.load_gather(mbuf, (tsplat,))
        rs = plsc.load_gather(rbuf, (tsplat,))
        for j in range(NJ):
          yj = ybuf[ti, pl.ds(16 * j, 16)]
          obuf[s, t, pl.ds(16 * j, 16)] = (yj - m) * rs * gvs[j] + bvs[j]
      return carry

    lax.fori_loop(0, CHUNK // 16, group, 0)

  def step(g, s, first):
    issue_tok(g + 2, (s + 2) % 3)
    wait_tok((s + 1) % 3)
    issue_add((s + 1) % 3)
    wait_add(s)
    if not first:
      out_copy(g - 3, s).wait()
    compute(s)
    out_copy(g, s).start()

  issue_tok(0, 0)
  issue_tok(1, 1)
  wait_tok(0)
  issue_add(0)

  def triple0(p, carry):
    for s3 in (0, 1, 2):
      step(3 * p + s3, s3, first=True)
    return carry

  def triple_rest(p, carry):
    for s3 in (0, 1, 2):
      step(3 * p + s3, s3, first=False)
    return carry

  triple0(0, 0)
  lax.fori_loop(1, TRIPLES, triple_rest, 0)

  g = NCHUNK - 2
  wait_tok(1)
  issue_add(1)
  wait_add(0)
  out_copy(g - 3, 0).wait()
  compute(0)
  out_copy(g, 0).start()

  g = NCHUNK - 1
  wait_add(1)
  out_copy(g - 3, 1).wait()
  compute(1)
  out_copy(g, 1).start()

  out_copy(NCHUNK - 3, 2).wait()
  out_copy(NCHUNK - 2, 0).wait()
  out_copy(NCHUNK - 1, 1).wait()


_sc_call = pl.kernel(
    _sc_body,
    out_type=jax.ShapeDtypeStruct((N_TOK, HIDDEN), jnp.float32),
    mesh=plsc.VectorSubcoreMesh(core_axis_name="c", subcore_axis_name="s"),
    compiler_params=pltpu.CompilerParams(needs_layout_passes=False),
    scratch_types=[
        pltpu.VMEM((3, CHUNK), jnp.int32),
        pltpu.VMEM((3, CHUNK), jnp.int32),
        pltpu.VMEM((3, CHUNK, HIDDEN), jnp.float32),
        pltpu.VMEM((3, CHUNK, HIDDEN), jnp.float32),
        pltpu.VMEM((HIDDEN,), jnp.float32),
        pltpu.VMEM((HIDDEN,), jnp.float32),
        pltpu.VMEM((16, HIDDEN), jnp.float32),
        pltpu.VMEM((16 * 17,), jnp.float32),
        pltpu.VMEM((16 * 17,), jnp.float32),
        pltpu.VMEM((16,), jnp.float32),
        pltpu.VMEM((16,), jnp.float32),
        pltpu.SemaphoreType.DMA((3,)),
        pltpu.SemaphoreType.DMA((3,)),
        pltpu.SemaphoreType.DMA((3,)),
    ],
)


def kernel(input_token, segment_ids, token_table, type_table, pos_table,
           gamma, beta):
  tok_idx = input_token.reshape(-1)
  cmb_idx = (2 * jnp.arange(MAX_LEN, dtype=jnp.int32)[None, :]
             + segment_ids).reshape(-1)
  combo = (pos_table[:, None, :] + type_table[None, :, :]).reshape(
      2 * MAX_LEN, HIDDEN)
  out = _sc_call(token_table, combo, tok_idx, cmb_idx, gamma, beta)
  return out.reshape(BATCH, MAX_LEN, HIDDEN)
